# baseline (speedup 1.0000x reference)
import functools

import jax
import jax.numpy as jnp
from jax import lax
from jax.experimental import pallas as pl
from jax.experimental.pallas import tpu as pltpu
from jax.experimental.pallas import tpu_sc as plsc

_T = 32
_W = 224
_G = 1792
_NW = 32
_GPW = _G // _NW
_NB = 2
_NBATCH = _GPW // _NB


def _sc_body(ls_hbm, x_hbm, o_hbm, ls_v, in_v, out_v, si0, si1, so0, so1):
    wid = lax.axis_index("s") * 2 + lax.axis_index("c")
    base = wid * _GPW
    pltpu.sync_copy(ls_hbm, ls_v)
    inv_ts = jnp.exp(ls_v[...] * -100.0)[0]
    sins = (si0, si1)
    souts = (so0, so1)

    def start_in(i, b):
        pltpu.async_copy(x_hbm.at[pl.ds(base + i * _NB, _NB)], in_v.at[b], sins[b])

    def wait_in(b):
        pltpu.make_async_copy(
            x_hbm.at[pl.ds(base, _NB)], in_v.at[b], sins[b]
        ).wait()

    def start_out(i, b):
        pltpu.async_copy(in_v.at[b], o_hbm.at[pl.ds(base + i * _NB, _NB)], souts[b])

    def wait_out(b):
        pltpu.make_async_copy(
            in_v.at[b], o_hbm.at[pl.ds(base, _NB)], souts[b]
        ).wait()

    def compute(b):
        @plsc.parallel_loop(0, _T, unroll=2)
        def trow(t):
            t_idx = jnp.minimum(t.astype(jnp.float32) * inv_ts, jnp.float32(_T - 1))
            low = t_idx.astype(jnp.int32)
            w = t_idx - low.astype(jnp.float32)
            up = jnp.minimum(low + 1, _T - 1)
            a = 1.0 - w
            for g in range(_NB):
                for k in range(_W // 16):
                    xl = in_v[b, g, low, pl.ds(k * 16, 16)]
                    xu = in_v[b, g, up, pl.ds(k * 16, 16)]
                    out_v[b, g, t, pl.ds(k * 16, 16)] = a * xl + w * xu

    def step(i, b):
        wait_in(b)

        @pl.when(i >= 2)
        def _():
            wait_out(b)

        start_out(i, b)

        @pl.when(i + 2 < _NBATCH)
        def _():
            start_in(i + 2, b)

    start_in(0, 0)
    start_in(1, 1)

    def pair(j, carry):
        step(j * 2, 0)
        step(j * 2 + 1, 1)
        return carry

    lax.fori_loop(0, _NBATCH // 2, pair, 0)
    wait_out(0)
    wait_out(1)


def kernel(x, log_timescale):
    B, H, W, T = x.shape
    xt = jnp.transpose(x, (0, 1, 3, 2)).reshape(_G, T, W)
    ls16 = jnp.broadcast_to(log_timescale, (16,))
    mesh = plsc.VectorSubcoreMesh(core_axis_name="c", subcore_axis_name="s")
    run = functools.partial(
        pl.kernel,
        mesh=mesh,
        out_type=jax.ShapeDtypeStruct((_G, T, W), jnp.float32),
        scratch_types=[
            pltpu.VMEM((16,), jnp.float32),
            pltpu.VMEM((2, _NB, _T, _W), jnp.float32),
            pltpu.VMEM((2, _NB, _T, _W), jnp.float32),
            pltpu.SemaphoreType.DMA,
            pltpu.SemaphoreType.DMA,
            pltpu.SemaphoreType.DMA,
            pltpu.SemaphoreType.DMA,
        ],
    )(_sc_body)
    out = run(ls16, xt)
    return jnp.transpose(out.reshape(B, H, T, W), (0, 1, 3, 2))


# speedup vs baseline: 1.6501x; 1.6501x over previous
import jax
import jax.numpy as jnp
from jax.experimental import pallas as pl
from jax.experimental.pallas import tpu as pltpu

_T = 32
_R = 448


def _mod_kernel(ls_ref, x_ref, o_ref):
    ls = ls_ref[0, 0]
    timescale = jnp.exp(ls * 100.0)
    trow = jax.lax.broadcasted_iota(jnp.int32, (_T, _T), 0)
    scol = jax.lax.broadcasted_iota(jnp.int32, (_T, _T), 1)
    t_idx = jnp.clip(trow.astype(jnp.float32) / timescale, 0.0, float(_T - 1))
    lower = jnp.floor(t_idx).astype(jnp.int32)
    upper = jnp.minimum(lower + 1, _T - 1)
    w = t_idx - lower.astype(jnp.float32)
    m = jnp.where(scol == lower, 1.0 - w, 0.0) + jnp.where(scol == upper, w, 0.0)
    for r in range(_R):
        o_ref[r] = jnp.dot(m, x_ref[r], preferred_element_type=jnp.float32)


def kernel(x, log_timescale):
    B, H, W, T = x.shape
    g = B * H
    xt = jnp.transpose(x, (0, 1, 3, 2)).reshape(g, T, W)
    grid = g // _R
    out = pl.pallas_call(
        _mod_kernel,
        grid=(grid,),
        in_specs=[
            pl.BlockSpec(memory_space=pltpu.SMEM),
            pl.BlockSpec((_R, T, W), lambda i: (i, 0, 0)),
        ],
        out_specs=pl.BlockSpec((_R, T, W), lambda i: (i, 0, 0)),
        out_shape=jax.ShapeDtypeStruct((g, T, W), jnp.float32),
        compiler_params=pltpu.CompilerParams(
            dimension_semantics=("arbitrary",),
        ),
    )(log_timescale.reshape(1, 1), xt)
    return jnp.transpose(out.reshape(B, H, T, W), (0, 1, 3, 2))
